# baseline (speedup 1.0000x reference)
import functools

import jax
import jax.numpy as jnp
from jax.experimental import pallas as pl
from jax.experimental.pallas import tpu as pltpu


def _mish(y):
    u = jnp.exp(jnp.minimum(y, 30.0))
    s = u * (u + 2.0)
    return y * (s / (s + 2.0))


def _ghost_kernel(x_ref, w1_ref, b1_ref, d_ref, b2_ref, o_ref, scr_ref, *,
                  B, C1, H, W):
    P = H * W
    PAD = 128

    scr_ref[:, :PAD] = jnp.zeros((C1, PAD), jnp.float32)
    scr_ref[:, PAD + P:] = jnp.zeros((C1, scr_ref.shape[1] - PAD - P),
                                     jnp.float32)

    w1 = w1_ref[...]
    b1 = b1_ref[...].astype(jnp.float32)
    dmat = d_ref[...]
    b2 = b2_ref[...].astype(jnp.float32)

    col = jax.lax.broadcasted_iota(jnp.int32, (1, P), 1) % W
    mask_l = col > 0
    mask_r = col < (W - 1)

    halves = ((0, P),) if P <= 384 else ((0, 384), (384, P - 384))

    for b in range(B):
        xb = x_ref[b]
        y = jnp.dot(w1, xb, preferred_element_type=jnp.float32) + b1
        x1 = _mish(y)
        o_ref[b, :C1] = x1
        scr_ref[:, PAD:PAD + P] = x1

        for lo, hw in halves:
            rstack = jnp.concatenate(
                [scr_ref[:, PAD + lo + dy * W - 1:
                         PAD + lo + dy * W - 1 + hw + 2]
                 for dy in (-1, 0, 1)], axis=0).astype(jnp.bfloat16)
            S = jnp.dot(dmat, rstack,
                        preferred_element_type=jnp.float32)
            tot = (S[C1:2 * C1, 1:hw + 1]
                   + jnp.where(mask_l[:, lo:lo + hw], S[:C1, :hw], 0.0)
                   + jnp.where(mask_r[:, lo:lo + hw], S[2 * C1:, 2:], 0.0)
                   + b2)
            x2 = _mish(tot)
            o_ref[b, C1:, lo:lo + hw] = x2


def kernel(x, w1, b1, w2, b2):
    N, Cin, H, W = x.shape
    C1 = w1.shape[0]
    P = H * W
    B = 2

    eye = jnp.eye(C1, dtype=jnp.float32)
    dmat = jnp.concatenate(
        [jnp.concatenate([eye * w2[:, dyi, dxi][:, None]
                          for dyi in range(3)], axis=1)
         for dxi in range(3)], axis=0).astype(jnp.bfloat16)

    out = pl.pallas_call(
        functools.partial(_ghost_kernel, B=B, C1=C1, H=H, W=W),
        out_shape=jax.ShapeDtypeStruct((N, 2 * C1, P), jnp.float32),
        grid=(N // B,),
        in_specs=[
            pl.BlockSpec((B, Cin, P), lambda i: (i, 0, 0)),
            pl.BlockSpec((C1, Cin), lambda i: (0, 0)),
            pl.BlockSpec((C1, 1), lambda i: (0, 0)),
            pl.BlockSpec((3 * C1, 3 * C1), lambda i: (0, 0)),
            pl.BlockSpec((C1, 1), lambda i: (0, 0)),
        ],
        out_specs=pl.BlockSpec((B, 2 * C1, P), lambda i: (i, 0, 0)),
        scratch_shapes=[pltpu.VMEM((C1, P + 2 * 128), jnp.float32)],
        compiler_params=pltpu.CompilerParams(
            dimension_semantics=("parallel",)),
    )(x.reshape(N, Cin, P).astype(jnp.bfloat16), w1.astype(jnp.bfloat16),
      b1.reshape(C1, 1),
      dmat, b2.reshape(C1, 1))
    return out.reshape(N, 2 * C1, H, W)


# speedup vs baseline: 1.0053x; 1.0053x over previous
import functools

import jax
import jax.numpy as jnp
from jax.experimental import pallas as pl
from jax.experimental.pallas import tpu as pltpu


def _mish(y):
    u = jnp.exp(jnp.minimum(y, 30.0))
    s = u * (u + 2.0)
    return y * (s / (s + 2.0))


def _ghost_kernel(x_ref, w1_ref, b1_ref, d_ref, b2_ref, o_ref, scr_ref, *,
                  B, C1, H, W):
    P = H * W
    PAD = 128

    scr_ref[:, :PAD] = jnp.zeros((C1, PAD), jnp.float32)
    scr_ref[:, PAD + P:] = jnp.zeros((C1, scr_ref.shape[1] - PAD - P),
                                     jnp.float32)

    w1 = w1_ref[...]
    b1 = b1_ref[...].astype(jnp.float32)
    dmat = d_ref[...]
    b2 = b2_ref[...].astype(jnp.float32)

    col = jax.lax.broadcasted_iota(jnp.int32, (1, P), 1) % W
    mask_l = col > 0
    mask_r = col < (W - 1)

    halves = ((0, P),) if P <= 384 else ((0, 384), (384, P - 384))

    for b in range(B):
        xb = x_ref[b].astype(jnp.bfloat16)
        y = jnp.dot(w1, xb, preferred_element_type=jnp.float32) + b1
        x1 = _mish(y)
        o_ref[b, :C1] = x1
        scr_ref[:, PAD:PAD + P] = x1

        for lo, hw in halves:
            rstack = jnp.concatenate(
                [scr_ref[:, PAD + lo + dy * W - 1:
                         PAD + lo + dy * W - 1 + hw + 2]
                 for dy in (-1, 0, 1)], axis=0).astype(jnp.bfloat16)
            S = jnp.dot(dmat, rstack,
                        preferred_element_type=jnp.float32)
            tot = (S[C1:2 * C1, 1:hw + 1]
                   + jnp.where(mask_l[:, lo:lo + hw], S[:C1, :hw], 0.0)
                   + jnp.where(mask_r[:, lo:lo + hw], S[2 * C1:, 2:], 0.0)
                   + b2)
            x2 = _mish(tot)
            o_ref[b, C1:, lo:lo + hw] = x2


def kernel(x, w1, b1, w2, b2):
    N, Cin, H, W = x.shape
    C1 = w1.shape[0]
    P = H * W
    B = 4

    eye = jnp.eye(C1, dtype=jnp.float32)
    dmat = jnp.concatenate(
        [jnp.concatenate([eye * w2[:, dyi, dxi][:, None]
                          for dyi in range(3)], axis=1)
         for dxi in range(3)], axis=0).astype(jnp.bfloat16)

    out = pl.pallas_call(
        functools.partial(_ghost_kernel, B=B, C1=C1, H=H, W=W),
        out_shape=jax.ShapeDtypeStruct((N, 2 * C1, P), x.dtype),
        grid=(N // B,),
        in_specs=[
            pl.BlockSpec((B, Cin, P), lambda i: (i, 0, 0)),
            pl.BlockSpec((C1, Cin), lambda i: (0, 0)),
            pl.BlockSpec((C1, 1), lambda i: (0, 0)),
            pl.BlockSpec((3 * C1, 3 * C1), lambda i: (0, 0)),
            pl.BlockSpec((C1, 1), lambda i: (0, 0)),
        ],
        out_specs=pl.BlockSpec((B, 2 * C1, P), lambda i: (i, 0, 0)),
        scratch_shapes=[pltpu.VMEM((C1, P + 2 * 128), jnp.float32)],
        compiler_params=pltpu.CompilerParams(
            dimension_semantics=("parallel",)),
    )(x.reshape(N, Cin, P), w1.astype(jnp.bfloat16), b1.reshape(C1, 1),
      dmat, b2.reshape(C1, 1))
    return out.reshape(N, 2 * C1, H, W)
